# baseline (speedup 1.0000x reference)
import functools
import math

import jax
import jax.numpy as jnp
from jax import lax
from jax.experimental import pallas as pl
from jax.experimental.pallas import tpu as pltpu
from jax.experimental.pallas import tpu_sc as plsc

_NUM_CLASSES = 1000
_TEMPERATURE = 0.5
_THRESHOLD = 0.95
_WARMUP = 1000
_ITERATION = 0
_BATCH = 16384
_NUM_SAMPLES = 1000000

_CA = (-math.log(_THRESHOLD)
       + (math.log(_NUM_CLASSES) + math.log(_THRESHOLD))
       * 0.5 * (1 + math.cos(_ITERATION / _WARMUP * math.pi)))
_TAU = math.exp(-_CA) if _ITERATION < _WARMUP else _THRESHOLD

_NW = 32
_HW = 1024
_LANES = 16
_MAIN = 31248
_TAIL = _NUM_SAMPLES - _NW * _MAIN
_RPW = _BATCH // _NW


def _sc_hist_body(y_hbm, out_hbm, yv, hv, htot):
    c = lax.axis_index("c")
    s = lax.axis_index("s")
    wid = s * 2 + c
    pltpu.sync_copy(y_hbm.at[pl.ds(wid * _MAIN, _MAIN)], yv.at[pl.ds(0, _MAIN)])

    @pl.when(wid == 0)
    def _():
        pltpu.sync_copy(y_hbm.at[pl.ds(_NW * _MAIN, _TAIL)],
                        yv.at[pl.ds(_MAIN, _TAIL)])

    zeros = jnp.zeros((16,), jnp.float32)

    def zbody(i, carry):
        hv[pl.ds(i * 16, 16)] = zeros
        return carry

    lax.fori_loop(0, _LANES * _HW // 16, zbody, 0)

    lane_off = lax.iota(jnp.int32, 16) * _HW
    ones = jnp.ones((16,), jnp.float32)

    def body(i, carry):
        v = yv[pl.ds(i * 16, 16)]
        v = jnp.minimum(jnp.maximum(v, 0), _HW - 1)
        plsc.addupdate_scatter(hv, [lane_off + v], ones)
        return carry

    lax.fori_loop(0, _MAIN // 16, body, 0)

    @pl.when(wid == 0)
    def _():
        lax.fori_loop(_MAIN // 16, (_MAIN + _TAIL) // 16, body, 0)

    def rbody(g, carry):
        acc = zeros
        for l in range(_LANES):
            acc = acc + hv[pl.ds(l * _HW + g * 16, 16)]
        htot[pl.ds(g * 16, 16)] = acc
        return carry

    lax.fori_loop(0, _HW // 16, rbody, 0)
    pltpu.sync_copy(htot, out_hbm.at[wid])


@functools.lru_cache(maxsize=1)
def _sc_hist():
    return pl.kernel(
        _sc_hist_body,
        out_type=jax.ShapeDtypeStruct((_NW, _HW), jnp.float32),
        mesh=plsc.VectorSubcoreMesh(core_axis_name="c", subcore_axis_name="s"),
        scratch_types=[
            pltpu.VMEM((_MAIN + _TAIL,), jnp.int32),
            pltpu.VMEM((_LANES * _HW,), jnp.float32),
            pltpu.VMEM((_HW,), jnp.float32),
        ],
        compiler_params=pltpu.CompilerParams(needs_layout_passes=False),
    )


_CB = 2048
_GRID = _BATCH // _CB


def _tc_stats_body(w_ref, s_ref, loss_ref, conf_ref, y_ref):
    w = w_ref[...]
    s = s_ref[...]
    inv_t = 1.0 / _TEMPERATURE

    mw = jnp.max(w, axis=0, keepdims=True)
    se = jnp.sum(jnp.exp((w - mw) * inv_t), axis=0, keepdims=True)
    conf_ref[...] = 1.0 / se

    iota = lax.broadcasted_iota(jnp.int32, (_NUM_CLASSES, _CB), 0)
    y = jnp.min(jnp.where(w == mw, iota, _NUM_CLASSES), axis=0, keepdims=True)
    y_ref[...] = y

    ms = jnp.max(s, axis=0, keepdims=True)
    lse = jnp.log(jnp.sum(jnp.exp(s - ms), axis=0, keepdims=True)) + ms
    loss_ref[...] = lse


_tc_stats = pl.pallas_call(
    _tc_stats_body,
    grid=(_GRID,),
    in_specs=[
        pl.BlockSpec((_NUM_CLASSES, _CB), lambda i: (0, i)),
        pl.BlockSpec((_NUM_CLASSES, _CB), lambda i: (0, i)),
    ],
    out_specs=[
        pl.BlockSpec((1, _CB), lambda i: (0, i)),
        pl.BlockSpec((1, _CB), lambda i: (0, i)),
        pl.BlockSpec((1, _CB), lambda i: (0, i)),
    ],
    out_shape=[
        jax.ShapeDtypeStruct((1, _BATCH), jnp.float32),
        jax.ShapeDtypeStruct((1, _BATCH), jnp.float32),
        jax.ShapeDtypeStruct((1, _BATCH), jnp.int32),
    ],
    compiler_params=pltpu.CompilerParams(
        dimension_semantics=("arbitrary",),
        vmem_limit_bytes=100 * 1024 * 1024,
    ),
)


def _sc_mask_body(parts_hbm, s_hbm, loss_hbm, conf_hbm, y_hbm, out_hbm,
                  ptv, lv, cv, yv, htab, idxv, pkv, pv, sem):
    c = lax.axis_index("c")
    s = lax.axis_index("s")
    wid = s * 2 + c

    pltpu.sync_copy(parts_hbm, ptv)
    pltpu.sync_copy(loss_hbm.at[pl.ds(wid * _RPW, _RPW)], lv)
    pltpu.sync_copy(conf_hbm.at[pl.ds(wid * _RPW, _RPW)], cv)
    pltpu.sync_copy(y_hbm.at[pl.ds(wid * _RPW, _RPW)], yv)

    lane = lax.iota(jnp.int32, 16)
    col0 = wid * _RPW

    def ibody(i, carry):
        yy = yv[pl.ds(i * 16, 16)]
        idxv[i // 8, pl.ds((i % 8) * 16, 16)] = yy * _BATCH + (col0 + i * 16 + lane)
        return carry

    lax.fori_loop(0, _RPW // 16, ibody, 0, unroll=8)
    copies = [
        pltpu.async_copy(s_hbm.at[idxv.at[j]], pkv.at[pl.ds(j * 128, 128)], sem)
        for j in range(_RPW // 128)
    ]
    for cp in copies:
        cp.wait()

    def rbody(g, macc):
        acc = jnp.zeros((16,), jnp.float32)
        for r in range(_NW):
            acc = acc + ptv[pl.ds(r * _HW + g * 16, 16)]
        htab[pl.ds(g * 16, 16)] = acc
        valid = (g * 16 + lane) < _NUM_CLASSES
        return jnp.maximum(macc, jnp.where(valid, acc, 0.0))

    macc = lax.fori_loop(0, _HW // 16, rbody, jnp.zeros((16,), jnp.float32))
    m2 = 2.0 * jnp.maximum(jnp.max(macc), 1.0)

    def mbody(i, acc):
        yy = yv[pl.ds(i * 16, 16)]
        hy = plsc.load_gather(htab, [yy])
        cf = cv[pl.ds(i * 16, 16)]
        ls = lv[pl.ds(i * 16, 16)] - pkv[pl.ds(i * 16, 16)]
        keep = cf * (m2 - hy) > _TAU * hy
        return acc + jnp.where(keep, ls, 0.0)

    acc = lax.fori_loop(0, _RPW // 16, mbody, jnp.zeros((16,), jnp.float32))
    pv[pl.ds(0, 16)] = acc
    pltpu.sync_copy(pv, out_hbm.at[wid])


@functools.lru_cache(maxsize=1)
def _sc_mask():
    return pl.kernel(
        _sc_mask_body,
        out_type=jax.ShapeDtypeStruct((_NW, 16), jnp.float32),
        mesh=plsc.VectorSubcoreMesh(core_axis_name="c", subcore_axis_name="s"),
        scratch_types=[
            pltpu.VMEM((_NW * _HW,), jnp.float32),
            pltpu.VMEM((_RPW,), jnp.float32),
            pltpu.VMEM((_RPW,), jnp.float32),
            pltpu.VMEM((_RPW,), jnp.int32),
            pltpu.VMEM((_HW,), jnp.float32),
            pltpu.VMEM((_RPW // 128, 128), jnp.int32),
            pltpu.VMEM((_RPW,), jnp.float32),
            pltpu.VMEM((16,), jnp.float32),
            pltpu.SemaphoreType.DMA,
        ],
        compiler_params=pltpu.CompilerParams(needs_layout_passes=False),
    )


def _tc_final_body(p_ref, out_ref):
    out_ref[0, 0] = jnp.sum(p_ref[...]) * (1.0 / _BATCH)


_tc_final = pl.pallas_call(
    _tc_final_body,
    in_specs=[pl.BlockSpec((_NW, 16), lambda: (0, 0))],
    out_specs=pl.BlockSpec(memory_space=pltpu.SMEM),
    out_shape=jax.ShapeDtypeStruct((1, 1), jnp.float32),
)


def kernel(logits_s, logits_w, Y_hat):
    parts = _sc_hist()(Y_hat)
    sT = logits_s.T
    lossraw, conf, y = _tc_stats(logits_w.T, sT)
    psums = _sc_mask()(parts.reshape(-1), sT.reshape(-1), lossraw.reshape(-1),
                       conf.reshape(-1), y.reshape(-1))
    return _tc_final(psums)[0, 0]


# speedup vs baseline: 1.4576x; 1.4576x over previous
import functools
import math

import jax
import jax.numpy as jnp
from jax import lax
from jax.experimental import pallas as pl
from jax.experimental.pallas import tpu as pltpu
from jax.experimental.pallas import tpu_sc as plsc

_NUM_CLASSES = 1000
_TEMPERATURE = 0.5
_THRESHOLD = 0.95
_WARMUP = 1000
_ITERATION = 0
_BATCH = 16384
_NUM_SAMPLES = 1000000

_CA = (-math.log(_THRESHOLD)
       + (math.log(_NUM_CLASSES) + math.log(_THRESHOLD))
       * 0.5 * (1 + math.cos(_ITERATION / _WARMUP * math.pi)))
_TAU = math.exp(-_CA) if _ITERATION < _WARMUP else _THRESHOLD

_NW = 32
_HW = 1024
_LANES = 16
_MAIN = 31248
_TAIL = _NUM_SAMPLES - _NW * _MAIN
_RPW = _BATCH // _NW


def _sc_hist_body(y_hbm, out_hbm, yv, hv, htot):
    c = lax.axis_index("c")
    s = lax.axis_index("s")
    wid = s * 2 + c
    pltpu.sync_copy(y_hbm.at[pl.ds(wid * _MAIN, _MAIN)], yv.at[pl.ds(0, _MAIN)])

    @pl.when(wid == 0)
    def _():
        pltpu.sync_copy(y_hbm.at[pl.ds(_NW * _MAIN, _TAIL)],
                        yv.at[pl.ds(_MAIN, _TAIL)])

    zeros = jnp.zeros((16,), jnp.float32)

    def zbody(i, carry):
        hv[pl.ds(i * 16, 16)] = zeros
        return carry

    lax.fori_loop(0, _LANES * _HW // 16, zbody, 0)

    lane_off = lax.iota(jnp.int32, 16) * _HW
    ones = jnp.ones((16,), jnp.float32)

    def body(i, carry):
        v = yv[pl.ds(i * 16, 16)]
        v = jnp.minimum(jnp.maximum(v, 0), _HW - 1)
        plsc.addupdate_scatter(hv, [lane_off + v], ones)
        return carry

    lax.fori_loop(0, _MAIN // 16, body, 0)

    @pl.when(wid == 0)
    def _():
        lax.fori_loop(_MAIN // 16, (_MAIN + _TAIL) // 16, body, 0)

    def rbody(g, carry):
        acc = zeros
        for l in range(_LANES):
            acc = acc + hv[pl.ds(l * _HW + g * 16, 16)]
        htot[pl.ds(g * 16, 16)] = acc
        return carry

    lax.fori_loop(0, _HW // 16, rbody, 0)
    pltpu.sync_copy(htot, out_hbm.at[wid])


@functools.lru_cache(maxsize=1)
def _sc_hist():
    return pl.kernel(
        _sc_hist_body,
        out_type=jax.ShapeDtypeStruct((_NW, _HW), jnp.float32),
        mesh=plsc.VectorSubcoreMesh(core_axis_name="c", subcore_axis_name="s"),
        scratch_types=[
            pltpu.VMEM((_MAIN + _TAIL,), jnp.int32),
            pltpu.VMEM((_LANES * _HW,), jnp.float32),
            pltpu.VMEM((_HW,), jnp.float32),
        ],
        compiler_params=pltpu.CompilerParams(needs_layout_passes=False),
    )


_CB = 2048
_GRID = _BATCH // _CB


def _tc_stats_body(w_ref, s_ref, loss_ref, conf_ref, y_ref):
    w = w_ref[...]
    s = s_ref[...]
    inv_t = 1.0 / _TEMPERATURE

    mw = jnp.max(w, axis=0, keepdims=True)
    se = jnp.sum(jnp.exp((w - mw) * inv_t), axis=0, keepdims=True)
    conf_ref[...] = 1.0 / se

    iota = lax.broadcasted_iota(jnp.int32, (_NUM_CLASSES, _CB), 0)
    y = jnp.min(jnp.where(w == mw, iota, _NUM_CLASSES), axis=0, keepdims=True)
    y_ref[...] = y
    sel = iota == y
    pick = jnp.sum(jnp.where(sel, s, 0.0), axis=0, keepdims=True)

    ms = jnp.max(s, axis=0, keepdims=True)
    lse = jnp.log(jnp.sum(jnp.exp(s - ms), axis=0, keepdims=True)) + ms
    loss_ref[...] = lse - pick


_tc_stats = pl.pallas_call(
    _tc_stats_body,
    grid=(_GRID,),
    in_specs=[
        pl.BlockSpec((_NUM_CLASSES, _CB), lambda i: (0, i)),
        pl.BlockSpec((_NUM_CLASSES, _CB), lambda i: (0, i)),
    ],
    out_specs=[
        pl.BlockSpec((1, _CB), lambda i: (0, i)),
        pl.BlockSpec((1, _CB), lambda i: (0, i)),
        pl.BlockSpec((1, _CB), lambda i: (0, i)),
    ],
    out_shape=[
        jax.ShapeDtypeStruct((1, _BATCH), jnp.float32),
        jax.ShapeDtypeStruct((1, _BATCH), jnp.float32),
        jax.ShapeDtypeStruct((1, _BATCH), jnp.int32),
    ],
    compiler_params=pltpu.CompilerParams(
        dimension_semantics=("arbitrary",),
        vmem_limit_bytes=100 * 1024 * 1024,
    ),
)


def _sc_mask_body(parts_hbm, loss_hbm, conf_hbm, y_hbm, out_hbm,
                  ptv, lv, cv, yv, htab, pv):
    c = lax.axis_index("c")
    s = lax.axis_index("s")
    wid = s * 2 + c

    pltpu.sync_copy(parts_hbm, ptv)
    pltpu.sync_copy(loss_hbm.at[pl.ds(wid * _RPW, _RPW)], lv)
    pltpu.sync_copy(conf_hbm.at[pl.ds(wid * _RPW, _RPW)], cv)
    pltpu.sync_copy(y_hbm.at[pl.ds(wid * _RPW, _RPW)], yv)

    lane = lax.iota(jnp.int32, 16)

    def rbody(g, macc):
        acc = jnp.zeros((16,), jnp.float32)
        for r in range(_NW):
            acc = acc + ptv[pl.ds(r * _HW + g * 16, 16)]
        htab[pl.ds(g * 16, 16)] = acc
        valid = (g * 16 + lane) < _NUM_CLASSES
        return jnp.maximum(macc, jnp.where(valid, acc, 0.0))

    macc = lax.fori_loop(0, _HW // 16, rbody, jnp.zeros((16,), jnp.float32))
    m2 = 2.0 * jnp.maximum(jnp.max(macc), 1.0)

    def mbody(i, acc):
        yy = yv[pl.ds(i * 16, 16)]
        hy = plsc.load_gather(htab, [yy])
        cf = cv[pl.ds(i * 16, 16)]
        ls = lv[pl.ds(i * 16, 16)]
        keep = cf * (m2 - hy) > _TAU * hy
        return acc + jnp.where(keep, ls, 0.0)

    acc = lax.fori_loop(0, _RPW // 16, mbody, jnp.zeros((16,), jnp.float32))
    pv[pl.ds(0, 16)] = acc
    pltpu.sync_copy(pv, out_hbm.at[wid])


@functools.lru_cache(maxsize=1)
def _sc_mask():
    return pl.kernel(
        _sc_mask_body,
        out_type=jax.ShapeDtypeStruct((_NW, 16), jnp.float32),
        mesh=plsc.VectorSubcoreMesh(core_axis_name="c", subcore_axis_name="s"),
        scratch_types=[
            pltpu.VMEM((_NW * _HW,), jnp.float32),
            pltpu.VMEM((_RPW,), jnp.float32),
            pltpu.VMEM((_RPW,), jnp.float32),
            pltpu.VMEM((_RPW,), jnp.int32),
            pltpu.VMEM((_HW,), jnp.float32),
            pltpu.VMEM((16,), jnp.float32),
        ],
        compiler_params=pltpu.CompilerParams(needs_layout_passes=False),
    )


def _tc_final_body(p_ref, out_ref):
    out_ref[0, 0] = jnp.sum(p_ref[...]) * (1.0 / _BATCH)


_tc_final = pl.pallas_call(
    _tc_final_body,
    in_specs=[pl.BlockSpec((_NW, 16), lambda: (0, 0))],
    out_specs=pl.BlockSpec(memory_space=pltpu.SMEM),
    out_shape=jax.ShapeDtypeStruct((1, 1), jnp.float32),
)


def kernel(logits_s, logits_w, Y_hat):
    parts = _sc_hist()(Y_hat)
    lossraw, conf, y = _tc_stats(logits_w.T, logits_s.T)
    psums = _sc_mask()(parts.reshape(-1), lossraw.reshape(-1),
                       conf.reshape(-1), y.reshape(-1))
    return _tc_final(psums)[0, 0]


# speedup vs baseline: 1.6162x; 1.1088x over previous
import functools
import math

import jax
import jax.numpy as jnp
from jax import lax
from jax.experimental import pallas as pl
from jax.experimental.pallas import tpu as pltpu
from jax.experimental.pallas import tpu_sc as plsc

_NUM_CLASSES = 1000
_TEMPERATURE = 0.5
_THRESHOLD = 0.95
_WARMUP = 1000
_ITERATION = 0
_BATCH = 16384
_NUM_SAMPLES = 1000000

_CA = (-math.log(_THRESHOLD)
       + (math.log(_NUM_CLASSES) + math.log(_THRESHOLD))
       * 0.5 * (1 + math.cos(_ITERATION / _WARMUP * math.pi)))
_TAU = math.exp(-_CA) if _ITERATION < _WARMUP else _THRESHOLD

_NW = 32
_HW = 1024
_LANES = 16
_MAIN = 31248
_TAIL = _NUM_SAMPLES - _NW * _MAIN
_RPW = _BATCH // _NW


def _sc_hist_body(y_hbm, out_hbm, yv, hv, htot):
    c = lax.axis_index("c")
    s = lax.axis_index("s")
    wid = s * 2 + c
    pltpu.sync_copy(y_hbm.at[pl.ds(wid * _MAIN, _MAIN)], yv.at[pl.ds(0, _MAIN)])

    @pl.when(wid == 0)
    def _():
        pltpu.sync_copy(y_hbm.at[pl.ds(_NW * _MAIN, _TAIL)],
                        yv.at[pl.ds(_MAIN, _TAIL)])

    zeros = jnp.zeros((16,), jnp.float32)

    def zbody(i, carry):
        hv[pl.ds(i * 16, 16)] = zeros
        return carry

    lax.fori_loop(0, _LANES * _HW // 16, zbody, 0)

    lane_off = lax.iota(jnp.int32, 16) * _HW
    ones = jnp.ones((16,), jnp.float32)

    def body(i, carry):
        v = yv[pl.ds(i * 16, 16)]
        v = jnp.minimum(jnp.maximum(v, 0), _HW - 1)
        plsc.addupdate_scatter(hv, [lane_off + v], ones)
        return carry

    lax.fori_loop(0, _MAIN // 16, body, 0)

    @pl.when(wid == 0)
    def _():
        lax.fori_loop(_MAIN // 16, (_MAIN + _TAIL) // 16, body, 0)

    def rbody(g, carry):
        acc = zeros
        for l in range(_LANES):
            acc = acc + hv[pl.ds(l * _HW + g * 16, 16)]
        htot[pl.ds(g * 16, 16)] = acc
        return carry

    lax.fori_loop(0, _HW // 16, rbody, 0)
    pltpu.sync_copy(htot, out_hbm.at[wid])


@functools.lru_cache(maxsize=1)
def _sc_hist():
    return pl.kernel(
        _sc_hist_body,
        out_type=jax.ShapeDtypeStruct((_NW, _HW), jnp.float32),
        mesh=plsc.VectorSubcoreMesh(core_axis_name="c", subcore_axis_name="s"),
        scratch_types=[
            pltpu.VMEM((_MAIN + _TAIL,), jnp.int32),
            pltpu.VMEM((_LANES * _HW,), jnp.float32),
            pltpu.VMEM((_HW,), jnp.float32),
        ],
        compiler_params=pltpu.CompilerParams(needs_layout_passes=False),
    )


_CB = 2048
_GRID = _BATCH // _CB


def _tc_stats_body(w_ref, s_ref, loss_ref, conf_ref, y_ref):
    w = w_ref[...]
    s = s_ref[...]
    inv_t = 1.0 / _TEMPERATURE

    ones_row = jnp.ones((1, _NUM_CLASSES), jnp.float32)
    dims = (((1,), (0,)), ((), ()))

    mw = jnp.max(w, axis=0, keepdims=True)
    ew = jnp.exp((w - mw) * inv_t)
    se = lax.dot_general(ones_row, ew, dims,
                         preferred_element_type=jnp.float32)
    conf_ref[...] = 1.0 / se

    iota = lax.broadcasted_iota(jnp.int32, (_NUM_CLASSES, _CB), 0)
    y = jnp.min(jnp.where(w == mw, iota, _NUM_CLASSES), axis=0, keepdims=True)
    y_ref[...] = y
    sel = iota == y
    pick = lax.dot_general(ones_row, jnp.where(sel, s, 0.0), dims,
                           preferred_element_type=jnp.float32)

    ms = jnp.max(s, axis=0, keepdims=True)
    es = jnp.exp(s - ms)
    ss = lax.dot_general(ones_row, es, dims,
                         preferred_element_type=jnp.float32)
    loss_ref[...] = jnp.log(ss) + ms - pick


_tc_stats = pl.pallas_call(
    _tc_stats_body,
    grid=(_GRID,),
    in_specs=[
        pl.BlockSpec((_NUM_CLASSES, _CB), lambda i: (0, i)),
        pl.BlockSpec((_NUM_CLASSES, _CB), lambda i: (0, i)),
    ],
    out_specs=[
        pl.BlockSpec((1, _CB), lambda i: (0, i)),
        pl.BlockSpec((1, _CB), lambda i: (0, i)),
        pl.BlockSpec((1, _CB), lambda i: (0, i)),
    ],
    out_shape=[
        jax.ShapeDtypeStruct((1, _BATCH), jnp.float32),
        jax.ShapeDtypeStruct((1, _BATCH), jnp.float32),
        jax.ShapeDtypeStruct((1, _BATCH), jnp.int32),
    ],
    compiler_params=pltpu.CompilerParams(
        dimension_semantics=("arbitrary",),
        vmem_limit_bytes=100 * 1024 * 1024,
    ),
)


def _sc_mask_body(parts_hbm, loss_hbm, conf_hbm, y_hbm, out_hbm,
                  ptv, lv, cv, yv, htab, pv):
    c = lax.axis_index("c")
    s = lax.axis_index("s")
    wid = s * 2 + c

    pltpu.sync_copy(parts_hbm, ptv)
    pltpu.sync_copy(loss_hbm.at[pl.ds(wid * _RPW, _RPW)], lv)
    pltpu.sync_copy(conf_hbm.at[pl.ds(wid * _RPW, _RPW)], cv)
    pltpu.sync_copy(y_hbm.at[pl.ds(wid * _RPW, _RPW)], yv)

    lane = lax.iota(jnp.int32, 16)

    def rbody(g, macc):
        acc = jnp.zeros((16,), jnp.float32)
        for r in range(_NW):
            acc = acc + ptv[pl.ds(r * _HW + g * 16, 16)]
        htab[pl.ds(g * 16, 16)] = acc
        valid = (g * 16 + lane) < _NUM_CLASSES
        return jnp.maximum(macc, jnp.where(valid, acc, 0.0))

    macc = lax.fori_loop(0, _HW // 16, rbody, jnp.zeros((16,), jnp.float32))
    m2 = 2.0 * jnp.maximum(jnp.max(macc), 1.0)

    def mbody(i, acc):
        yy = yv[pl.ds(i * 16, 16)]
        hy = plsc.load_gather(htab, [yy])
        cf = cv[pl.ds(i * 16, 16)]
        ls = lv[pl.ds(i * 16, 16)]
        keep = cf * (m2 - hy) > _TAU * hy
        return acc + jnp.where(keep, ls, 0.0)

    acc = lax.fori_loop(0, _RPW // 16, mbody, jnp.zeros((16,), jnp.float32))
    pv[pl.ds(0, 16)] = acc
    pltpu.sync_copy(pv, out_hbm.at[wid])


@functools.lru_cache(maxsize=1)
def _sc_mask():
    return pl.kernel(
        _sc_mask_body,
        out_type=jax.ShapeDtypeStruct((_NW, 16), jnp.float32),
        mesh=plsc.VectorSubcoreMesh(core_axis_name="c", subcore_axis_name="s"),
        scratch_types=[
            pltpu.VMEM((_NW * _HW,), jnp.float32),
            pltpu.VMEM((_RPW,), jnp.float32),
            pltpu.VMEM((_RPW,), jnp.float32),
            pltpu.VMEM((_RPW,), jnp.int32),
            pltpu.VMEM((_HW,), jnp.float32),
            pltpu.VMEM((16,), jnp.float32),
        ],
        compiler_params=pltpu.CompilerParams(needs_layout_passes=False),
    )


def _tc_final_body(p_ref, out_ref):
    out_ref[0, 0] = jnp.sum(p_ref[...]) * (1.0 / _BATCH)


_tc_final = pl.pallas_call(
    _tc_final_body,
    in_specs=[pl.BlockSpec((_NW, 16), lambda: (0, 0))],
    out_specs=pl.BlockSpec(memory_space=pltpu.SMEM),
    out_shape=jax.ShapeDtypeStruct((1, 1), jnp.float32),
)


def kernel(logits_s, logits_w, Y_hat):
    parts = _sc_hist()(Y_hat)
    lossraw, conf, y = _tc_stats(logits_w.T, logits_s.T)
    psums = _sc_mask()(parts.reshape(-1), lossraw.reshape(-1),
                       conf.reshape(-1), y.reshape(-1))
    return _tc_final(psums)[0, 0]


# speedup vs baseline: 1.6829x; 1.0413x over previous
import functools
import math

import jax
import jax.numpy as jnp
from jax import lax
from jax.experimental import pallas as pl
from jax.experimental.pallas import tpu as pltpu
from jax.experimental.pallas import tpu_sc as plsc

_NUM_CLASSES = 1000
_TEMPERATURE = 0.5
_THRESHOLD = 0.95
_WARMUP = 1000
_ITERATION = 0
_BATCH = 16384
_NUM_SAMPLES = 1000000

_CA = (-math.log(_THRESHOLD)
       + (math.log(_NUM_CLASSES) + math.log(_THRESHOLD))
       * 0.5 * (1 + math.cos(_ITERATION / _WARMUP * math.pi)))
_TAU = math.exp(-_CA) if _ITERATION < _WARMUP else _THRESHOLD

_NW = 32
_HW = 1024
_LANES = 16
_MAIN = 31248
_TAIL = _NUM_SAMPLES - _NW * _MAIN
_RPW = _BATCH // _NW


def _sc_hist_body(y_hbm, out_hbm, yv, hv, htot):
    c = lax.axis_index("c")
    s = lax.axis_index("s")
    wid = s * 2 + c
    pltpu.sync_copy(y_hbm.at[pl.ds(wid * _MAIN, _MAIN)], yv.at[pl.ds(0, _MAIN)])

    @pl.when(wid == 0)
    def _():
        pltpu.sync_copy(y_hbm.at[pl.ds(_NW * _MAIN, _TAIL)],
                        yv.at[pl.ds(_MAIN, _TAIL)])

    zeros = jnp.zeros((16,), jnp.float32)

    def zbody(i, carry):
        hv[pl.ds(i * 16, 16)] = zeros
        return carry

    lax.fori_loop(0, _LANES * _HW // 16, zbody, 0)

    lane_off = lax.iota(jnp.int32, 16) * _HW
    ones = jnp.ones((16,), jnp.float32)

    def body(i, carry):
        v = yv[pl.ds(i * 16, 16)]
        v = jnp.minimum(jnp.maximum(v, 0), _HW - 1)
        plsc.addupdate_scatter(hv, [lane_off + v], ones)
        return carry

    lax.fori_loop(0, _MAIN // 16, body, 0)

    @pl.when(wid == 0)
    def _():
        lax.fori_loop(_MAIN // 16, (_MAIN + _TAIL) // 16, body, 0)

    def rbody(g, carry):
        acc = zeros
        for l in range(_LANES):
            acc = acc + hv[pl.ds(l * _HW + g * 16, 16)]
        htot[pl.ds(g * 16, 16)] = acc
        return carry

    lax.fori_loop(0, _HW // 16, rbody, 0)
    pltpu.sync_copy(htot, out_hbm.at[pl.ds(wid * _HW, _HW)])


@functools.lru_cache(maxsize=1)
def _sc_hist():
    return pl.kernel(
        _sc_hist_body,
        out_type=jax.ShapeDtypeStruct((_NW * _HW,), jnp.float32),
        mesh=plsc.VectorSubcoreMesh(core_axis_name="c", subcore_axis_name="s"),
        scratch_types=[
            pltpu.VMEM((_MAIN + _TAIL,), jnp.int32),
            pltpu.VMEM((_LANES * _HW,), jnp.float32),
            pltpu.VMEM((_HW,), jnp.float32),
        ],
        compiler_params=pltpu.CompilerParams(needs_layout_passes=False),
    )


_CB = 2048
_GRID = _BATCH // _CB


def _tc_stats_body(w_ref, s_ref, loss_ref, conf_ref, y_ref):
    w = w_ref[...]
    s = s_ref[...]
    inv_t = 1.0 / _TEMPERATURE

    ones_row = jnp.ones((1, _NUM_CLASSES), jnp.float32)
    dims = (((1,), (0,)), ((), ()))

    mw = jnp.max(w, axis=0, keepdims=True)
    ew = jnp.exp((w - mw) * inv_t)
    se = lax.dot_general(ones_row, ew, dims,
                         preferred_element_type=jnp.float32)
    conf_ref[...] = 1.0 / se

    iota = lax.broadcasted_iota(jnp.int32, (_NUM_CLASSES, _CB), 0)
    y = jnp.min(jnp.where(w == mw, iota, _NUM_CLASSES), axis=0, keepdims=True)
    y_ref[...] = y
    sel = iota == y
    pick = lax.dot_general(ones_row, jnp.where(sel, s, 0.0), dims,
                           preferred_element_type=jnp.float32)

    ms = jnp.max(s, axis=0, keepdims=True)
    es = jnp.exp(s - ms)
    ss = lax.dot_general(ones_row, es, dims,
                         preferred_element_type=jnp.float32)
    loss_ref[...] = jnp.log(ss) + ms - pick


_tc_stats = pl.pallas_call(
    _tc_stats_body,
    grid=(_GRID,),
    in_specs=[
        pl.BlockSpec((_NUM_CLASSES, _CB), lambda i: (0, i)),
        pl.BlockSpec((_NUM_CLASSES, _CB), lambda i: (0, i)),
    ],
    out_specs=[
        pl.BlockSpec((1, _CB), lambda i: (0, i)),
        pl.BlockSpec((1, _CB), lambda i: (0, i)),
        pl.BlockSpec((1, _CB), lambda i: (0, i)),
    ],
    out_shape=[
        jax.ShapeDtypeStruct((1, _BATCH), jnp.float32),
        jax.ShapeDtypeStruct((1, _BATCH), jnp.float32),
        jax.ShapeDtypeStruct((1, _BATCH), jnp.int32),
    ],
    compiler_params=pltpu.CompilerParams(
        dimension_semantics=("arbitrary",),
        vmem_limit_bytes=100 * 1024 * 1024,
    ),
)


def _sc_mask_body(parts_hbm, loss_hbm, conf_hbm, y_hbm, out_hbm,
                  ptv, lv, cv, yv, htab, pv):
    c = lax.axis_index("c")
    s = lax.axis_index("s")
    wid = s * 2 + c

    pltpu.sync_copy(parts_hbm, ptv)
    pltpu.sync_copy(loss_hbm.at[pl.ds(wid * _RPW, _RPW)], lv)
    pltpu.sync_copy(conf_hbm.at[pl.ds(wid * _RPW, _RPW)], cv)
    pltpu.sync_copy(y_hbm.at[pl.ds(wid * _RPW, _RPW)], yv)

    lane = lax.iota(jnp.int32, 16)

    def rbody(g, macc):
        acc = jnp.zeros((16,), jnp.float32)
        for r in range(_NW):
            acc = acc + ptv[pl.ds(r * _HW + g * 16, 16)]
        htab[pl.ds(g * 16, 16)] = acc
        valid = (g * 16 + lane) < _NUM_CLASSES
        return jnp.maximum(macc, jnp.where(valid, acc, 0.0))

    macc = lax.fori_loop(0, _HW // 16, rbody, jnp.zeros((16,), jnp.float32))
    m2 = 2.0 * jnp.maximum(jnp.max(macc), 1.0)

    def mbody(i, acc):
        yy = yv[pl.ds(i * 16, 16)]
        hy = plsc.load_gather(htab, [yy])
        cf = cv[pl.ds(i * 16, 16)]
        ls = lv[pl.ds(i * 16, 16)]
        keep = cf * (m2 - hy) > _TAU * hy
        return acc + jnp.where(keep, ls, 0.0)

    acc = lax.fori_loop(0, _RPW // 16, mbody, jnp.zeros((16,), jnp.float32))
    pv[pl.ds(0, 16)] = acc
    pltpu.sync_copy(pv, out_hbm.at[wid])


@functools.lru_cache(maxsize=1)
def _sc_mask():
    return pl.kernel(
        _sc_mask_body,
        out_type=jax.ShapeDtypeStruct((_NW, 16), jnp.float32),
        mesh=plsc.VectorSubcoreMesh(core_axis_name="c", subcore_axis_name="s"),
        scratch_types=[
            pltpu.VMEM((_NW * _HW,), jnp.float32),
            pltpu.VMEM((_RPW,), jnp.float32),
            pltpu.VMEM((_RPW,), jnp.float32),
            pltpu.VMEM((_RPW,), jnp.int32),
            pltpu.VMEM((_HW,), jnp.float32),
            pltpu.VMEM((16,), jnp.float32),
        ],
        compiler_params=pltpu.CompilerParams(needs_layout_passes=False),
    )


def _tc_final_body(p_ref, out_ref):
    out_ref[0, 0] = jnp.sum(p_ref[...]) * (1.0 / _BATCH)


_tc_final = pl.pallas_call(
    _tc_final_body,
    in_specs=[pl.BlockSpec((_NW, 16), lambda: (0, 0))],
    out_specs=pl.BlockSpec(memory_space=pltpu.SMEM),
    out_shape=jax.ShapeDtypeStruct((1, 1), jnp.float32),
)


def kernel(logits_s, logits_w, Y_hat):
    lossraw, conf, y = _tc_stats(logits_w.T, logits_s.T)
    parts = _sc_hist()(Y_hat)
    psums = _sc_mask()(parts, lossraw.reshape(-1),
                       conf.reshape(-1), y.reshape(-1))
    return _tc_final(psums)[0, 0]


# speedup vs baseline: 1.7427x; 1.0355x over previous
import functools
import math

import jax
import jax.numpy as jnp
from jax import lax
from jax.experimental import pallas as pl
from jax.experimental.pallas import tpu as pltpu
from jax.experimental.pallas import tpu_sc as plsc

_NUM_CLASSES = 1000
_TEMPERATURE = 0.5
_THRESHOLD = 0.95
_WARMUP = 1000
_ITERATION = 0
_BATCH = 16384
_NUM_SAMPLES = 1000000

_CA = (-math.log(_THRESHOLD)
       + (math.log(_NUM_CLASSES) + math.log(_THRESHOLD))
       * 0.5 * (1 + math.cos(_ITERATION / _WARMUP * math.pi)))
_TAU = math.exp(-_CA) if _ITERATION < _WARMUP else _THRESHOLD

_NW = 32
_HW = 1024
_LANES = 16
_MAIN = 31248
_TAIL = _NUM_SAMPLES - _NW * _MAIN
_RPW = _BATCH // _NW


def _sc_hist_body(y_hbm, out_hbm, yv, hv, htot):
    c = lax.axis_index("c")
    s = lax.axis_index("s")
    wid = s * 2 + c
    pltpu.sync_copy(y_hbm.at[pl.ds(wid * _MAIN, _MAIN)], yv.at[pl.ds(0, _MAIN)])

    @pl.when(wid == 0)
    def _():
        pltpu.sync_copy(y_hbm.at[pl.ds(_NW * _MAIN, _TAIL)],
                        yv.at[pl.ds(_MAIN, _TAIL)])

    zeros = jnp.zeros((16,), jnp.float32)

    def zbody(i, carry):
        hv[pl.ds(i * 16, 16)] = zeros
        return carry

    lax.fori_loop(0, _LANES * _HW // 16, zbody, 0)

    lane_off = lax.iota(jnp.int32, 16) * _HW
    ones = jnp.ones((16,), jnp.float32)

    def body(i, carry):
        v = yv[pl.ds(i * 16, 16)]
        v = jnp.minimum(jnp.maximum(v, 0), _HW - 1)
        plsc.addupdate_scatter(hv, [lane_off + v], ones)
        return carry

    lax.fori_loop(0, _MAIN // 16, body, 0)

    @pl.when(wid == 0)
    def _():
        lax.fori_loop(_MAIN // 16, (_MAIN + _TAIL) // 16, body, 0)

    def rbody(g, carry):
        acc = zeros
        for l in range(_LANES):
            acc = acc + hv[pl.ds(l * _HW + g * 16, 16)]
        htot[pl.ds(g * 16, 16)] = acc
        return carry

    lax.fori_loop(0, _HW // 16, rbody, 0)
    pltpu.sync_copy(htot, out_hbm.at[pl.ds(wid * _HW, _HW)])


@functools.lru_cache(maxsize=1)
def _sc_hist():
    return pl.kernel(
        _sc_hist_body,
        out_type=jax.ShapeDtypeStruct((_NW * _HW,), jnp.float32),
        mesh=plsc.VectorSubcoreMesh(core_axis_name="c", subcore_axis_name="s"),
        scratch_types=[
            pltpu.VMEM((_MAIN + _TAIL,), jnp.int32),
            pltpu.VMEM((_LANES * _HW,), jnp.float32),
            pltpu.VMEM((_HW,), jnp.float32),
        ],
        compiler_params=pltpu.CompilerParams(needs_layout_passes=False),
    )


_CB = 2048
_GRID = _BATCH // _CB


def _tc_stats_body(w_ref, s_ref, loss_ref, conf_ref, y_ref):
    w = w_ref[...]
    s = s_ref[...]
    inv_t = 1.0 / _TEMPERATURE

    ones_row = jnp.ones((1, _NUM_CLASSES), jnp.float32)
    dims = (((1,), (0,)), ((), ()))

    mw = jnp.max(w, axis=0, keepdims=True)
    ew = jnp.exp((w - mw) * inv_t)
    se = lax.dot_general(ones_row, ew, dims,
                         preferred_element_type=jnp.float32)
    conf_ref[...] = 1.0 / se

    iota = lax.broadcasted_iota(jnp.int32, (_NUM_CLASSES, _CB), 0)
    y = jnp.min(jnp.where(w == mw, iota, _NUM_CLASSES), axis=0, keepdims=True)
    y_ref[...] = y
    sel = iota == y
    pick = lax.dot_general(ones_row, jnp.where(sel, s, 0.0), dims,
                           preferred_element_type=jnp.float32)

    ms = jnp.max(s, axis=0, keepdims=True)
    es = jnp.exp(s - ms)
    ss = lax.dot_general(ones_row, es, dims,
                         preferred_element_type=jnp.float32)
    loss_ref[...] = jnp.log(ss) + ms - pick


_tc_stats = pl.pallas_call(
    _tc_stats_body,
    grid=(_GRID,),
    in_specs=[
        pl.BlockSpec((_NUM_CLASSES, _CB), lambda i: (0, i)),
        pl.BlockSpec((_NUM_CLASSES, _CB), lambda i: (0, i)),
    ],
    out_specs=[
        pl.BlockSpec((1, _CB), lambda i: (0, i)),
        pl.BlockSpec((1, _CB), lambda i: (0, i)),
        pl.BlockSpec((1, _CB), lambda i: (0, i)),
    ],
    out_shape=[
        jax.ShapeDtypeStruct((1, _BATCH), jnp.float32),
        jax.ShapeDtypeStruct((1, _BATCH), jnp.float32),
        jax.ShapeDtypeStruct((1, _BATCH), jnp.int32),
    ],
    compiler_params=pltpu.CompilerParams(
        dimension_semantics=("arbitrary",),
        vmem_limit_bytes=100 * 1024 * 1024,
    ),
)


_MB = 2048
_MGRID = _BATCH // _MB


def _tc_mask_body(parts_ref, loss_ref, conf_ref, y_ref, out_ref):
    pid = pl.program_id(0)

    @pl.when(pid == 0)
    def _():
        out_ref[0, 0] = 0.0

    h_row = lax.dot_general(jnp.ones((1, _NW), jnp.float32), parts_ref[...],
                            (((1,), (0,)), ((), ())),
                            preferred_element_type=jnp.float32)
    col = lax.broadcasted_iota(jnp.int32, (1, _HW), 1)
    m2 = 2.0 * jnp.maximum(jnp.max(jnp.where(col < _NUM_CLASSES, h_row, 0.0)), 1.0)

    y = y_ref[...]
    iota0 = lax.broadcasted_iota(jnp.int32, (_HW, _MB), 0)
    onehot = jnp.where(iota0 == y, 1.0, 0.0)
    hy = lax.dot_general(h_row, onehot, (((1,), (0,)), ((), ())),
                         preferred_element_type=jnp.float32)

    conf = conf_ref[...]
    keep = conf * (m2 - hy) > _TAU * hy
    psum = jnp.sum(jnp.where(keep, loss_ref[...], 0.0))
    out_ref[0, 0] += psum * (1.0 / _BATCH)


_tc_mask = pl.pallas_call(
    _tc_mask_body,
    grid=(_MGRID,),
    in_specs=[
        pl.BlockSpec((_NW, _HW), lambda i: (0, 0)),
        pl.BlockSpec((1, _MB), lambda i: (0, i)),
        pl.BlockSpec((1, _MB), lambda i: (0, i)),
        pl.BlockSpec((1, _MB), lambda i: (0, i)),
    ],
    out_specs=pl.BlockSpec(memory_space=pltpu.SMEM),
    out_shape=jax.ShapeDtypeStruct((1, 1), jnp.float32),
    compiler_params=pltpu.CompilerParams(
        dimension_semantics=("arbitrary",),
    ),
)


def kernel(logits_s, logits_w, Y_hat):
    lossraw, conf, y = _tc_stats(logits_w.T, logits_s.T)
    parts = _sc_hist()(Y_hat)
    return _tc_mask(parts.reshape(_NW, _HW), lossraw, conf, y)[0, 0]


# speedup vs baseline: 1.7756x; 1.0189x over previous
import functools
import math

import jax
import jax.numpy as jnp
from jax import lax
from jax.experimental import pallas as pl
from jax.experimental.pallas import tpu as pltpu
from jax.experimental.pallas import tpu_sc as plsc

_NUM_CLASSES = 1000
_TEMPERATURE = 0.5
_THRESHOLD = 0.95
_WARMUP = 1000
_ITERATION = 0
_BATCH = 16384
_NUM_SAMPLES = 1000000

_CA = (-math.log(_THRESHOLD)
       + (math.log(_NUM_CLASSES) + math.log(_THRESHOLD))
       * 0.5 * (1 + math.cos(_ITERATION / _WARMUP * math.pi)))
_TAU = math.exp(-_CA) if _ITERATION < _WARMUP else _THRESHOLD

_NW = 32
_HW = 1024
_LANES = 16
_MAIN = 31248
_TAIL = _NUM_SAMPLES - _NW * _MAIN
_RPW = _BATCH // _NW


def _sc_hist_body(y_hbm, out_hbm, yv, hv, htot):
    c = lax.axis_index("c")
    s = lax.axis_index("s")
    wid = s * 2 + c
    pltpu.sync_copy(y_hbm.at[pl.ds(wid * _MAIN, _MAIN)], yv.at[pl.ds(0, _MAIN)])

    @pl.when(wid == 0)
    def _():
        pltpu.sync_copy(y_hbm.at[pl.ds(_NW * _MAIN, _TAIL)],
                        yv.at[pl.ds(_MAIN, _TAIL)])

    zeros = jnp.zeros((16,), jnp.float32)

    def zbody(i, carry):
        hv[pl.ds(i * 16, 16)] = zeros
        return carry

    lax.fori_loop(0, _LANES * _HW // 16, zbody, 0)

    lane_off = lax.iota(jnp.int32, 16) * _HW
    ones = jnp.ones((16,), jnp.float32)

    def body(i, carry):
        v = yv[pl.ds(i * 16, 16)]
        v = jnp.minimum(jnp.maximum(v, 0), _HW - 1)
        plsc.addupdate_scatter(hv, [lane_off + v], ones)
        return carry

    lax.fori_loop(0, _MAIN // 16, body, 0)

    @pl.when(wid == 0)
    def _():
        lax.fori_loop(_MAIN // 16, (_MAIN + _TAIL) // 16, body, 0)

    def rbody(g, carry):
        acc = zeros
        for l in range(_LANES):
            acc = acc + hv[pl.ds(l * _HW + g * 16, 16)]
        htot[pl.ds(g * 16, 16)] = acc
        return carry

    lax.fori_loop(0, _HW // 16, rbody, 0)
    pltpu.sync_copy(htot, out_hbm.at[wid])


@functools.lru_cache(maxsize=1)
def _sc_hist():
    return pl.kernel(
        _sc_hist_body,
        out_type=jax.ShapeDtypeStruct((_NW, _HW), jnp.float32),
        mesh=plsc.VectorSubcoreMesh(core_axis_name="c", subcore_axis_name="s"),
        scratch_types=[
            pltpu.VMEM((_MAIN + _TAIL,), jnp.int32),
            pltpu.VMEM((_LANES * _HW,), jnp.float32),
            pltpu.VMEM((_HW,), jnp.float32),
        ],
        compiler_params=pltpu.CompilerParams(needs_layout_passes=False),
    )


_CB = 2048
_GRID = _BATCH // _CB


def _tc_stats_body(w_ref, s_ref, loss_ref, conf_ref, y_ref):
    w = w_ref[...]
    s = s_ref[...]
    inv_t = 1.0 / _TEMPERATURE

    ones_row = jnp.ones((1, _NUM_CLASSES), jnp.float32)
    dims = (((1,), (0,)), ((), ()))

    mw = jnp.max(w, axis=0, keepdims=True)
    ew = jnp.exp((w - mw) * inv_t)
    se = lax.dot_general(ones_row, ew, dims,
                         preferred_element_type=jnp.float32)
    conf_ref[...] = 1.0 / se

    iota = lax.broadcasted_iota(jnp.int32, (_NUM_CLASSES, _CB), 0)
    y = jnp.min(jnp.where(w == mw, iota, _NUM_CLASSES), axis=0, keepdims=True)
    y_ref[...] = y
    sel = iota == y
    pick = lax.dot_general(ones_row, jnp.where(sel, s, 0.0), dims,
                           preferred_element_type=jnp.float32)

    ms = jnp.max(s, axis=0, keepdims=True)
    es = jnp.exp(s - ms)
    ss = lax.dot_general(ones_row, es, dims,
                         preferred_element_type=jnp.float32)
    loss_ref[...] = jnp.log(ss) + ms - pick


_tc_stats = pl.pallas_call(
    _tc_stats_body,
    grid=(_GRID,),
    in_specs=[
        pl.BlockSpec((_NUM_CLASSES, _CB), lambda i: (0, i)),
        pl.BlockSpec((_NUM_CLASSES, _CB), lambda i: (0, i)),
    ],
    out_specs=[
        pl.BlockSpec((1, _CB), lambda i: (0, i)),
        pl.BlockSpec((1, _CB), lambda i: (0, i)),
        pl.BlockSpec((1, _CB), lambda i: (0, i)),
    ],
    out_shape=[
        jax.ShapeDtypeStruct((1, _BATCH), jnp.float32),
        jax.ShapeDtypeStruct((1, _BATCH), jnp.float32),
        jax.ShapeDtypeStruct((1, _BATCH), jnp.int32),
    ],
    compiler_params=pltpu.CompilerParams(
        dimension_semantics=("arbitrary",),
        vmem_limit_bytes=100 * 1024 * 1024,
    ),
)


_MB = 2048
_MGRID = _BATCH // _MB


def _tc_mask_body(parts_ref, loss_ref, conf_ref, y_ref, out_ref):
    pid = pl.program_id(0)

    @pl.when(pid == 0)
    def _():
        out_ref[0, 0] = 0.0

    h_row = lax.dot_general(jnp.ones((1, _NW), jnp.float32), parts_ref[...],
                            (((1,), (0,)), ((), ())),
                            preferred_element_type=jnp.float32)
    col = lax.broadcasted_iota(jnp.int32, (1, _HW), 1)
    m2 = 2.0 * jnp.maximum(jnp.max(jnp.where(col < _NUM_CLASSES, h_row, 0.0)), 1.0)

    y = y_ref[...]
    iota0 = lax.broadcasted_iota(jnp.int32, (_HW, _MB), 0)
    onehot = jnp.where(iota0 == y, 1.0, 0.0)
    hy = lax.dot_general(h_row, onehot, (((1,), (0,)), ((), ())),
                         preferred_element_type=jnp.float32)

    conf = conf_ref[...]
    keep = conf * (m2 - hy) > _TAU * hy
    psum = jnp.sum(jnp.where(keep, loss_ref[...], 0.0))
    out_ref[0, 0] += psum * (1.0 / _BATCH)


_tc_mask = pl.pallas_call(
    _tc_mask_body,
    grid=(_MGRID,),
    in_specs=[
        pl.BlockSpec((_NW, _HW), lambda i: (0, 0)),
        pl.BlockSpec((1, _MB), lambda i: (0, i)),
        pl.BlockSpec((1, _MB), lambda i: (0, i)),
        pl.BlockSpec((1, _MB), lambda i: (0, i)),
    ],
    out_specs=pl.BlockSpec(memory_space=pltpu.SMEM),
    out_shape=jax.ShapeDtypeStruct((1, 1), jnp.float32),
    compiler_params=pltpu.CompilerParams(
        dimension_semantics=("arbitrary",),
    ),
)


def kernel(logits_s, logits_w, Y_hat):
    lossraw, conf, y = _tc_stats(logits_w.T, logits_s.T)
    parts = _sc_hist()(Y_hat)
    return _tc_mask(parts, lossraw, conf, y)[0, 0]
